# baseline (speedup 1.0000x reference)
import functools

import jax
import jax.numpy as jnp
from jax import lax
from jax.experimental import pallas as pl
from jax.experimental.pallas import tpu as pltpu
from jax.experimental.pallas import tpu_sc as plsc

K = 16
R = 640
CT = 512
RO = 512
BIG = 1e9
NSC = 32
CHR = 8


def _bf(v):
    return v.astype(jnp.bfloat16)


def _proj_body(x_ref, ws_ref, bs_ref, wh_ref, bh_ref, s_ref, h_ref, sq_ref):
    xb = _bf(x_ref[...])
    s = jax.lax.dot_general(xb, _bf(ws_ref[...]), (((1,), (0,)), ((), ())),
                            preferred_element_type=jnp.float32) + bs_ref[...]
    h = jax.lax.dot_general(xb, _bf(wh_ref[...]), (((1,), (0,)), ((), ())),
                            preferred_element_type=jnp.float32) + bh_ref[...]
    s_ref[...] = s
    h_ref[...] = jnp.concatenate(
        [h, jnp.zeros((h.shape[0], 128 - h.shape[1]), jnp.float32)], axis=1)
    sq_ref[...] = jnp.sum(s * s, axis=1, keepdims=True)


def _knn_body(c0_ref, nt_ref, srow_ref, sqrow_ref, birow_ref,
              sT_ref, sqT_ref, biT_ref, giota_ref,
              idxo_ref, wo_ref, dmat_ref, *, S):
    i = pl.program_id(0)
    c0 = c0_ref[i]
    nt = nt_ref[i]

    s_row = _bf(srow_ref[...]).astype(jnp.float32)
    sq_row = sqrow_ref[...]
    bi_row = birow_ref[...]

    def p1(t, carry):
        c = pl.multiple_of(t * CT, CT)
        gc = pl.multiple_of(c0 + c, CT)
        scT = _bf(sT_ref[:, pl.ds(gc, CT)]).astype(jnp.float32)
        sqc = sqT_ref[:, pl.ds(gc, CT)]
        bic = biT_ref[:, pl.ds(gc, CT)]
        acc = s_row[:, 0:1] * scT[0:1, :]
        for a in range(1, S):
            acc = acc + s_row[:, a:a + 1] * scT[a:a + 1, :]
        d = sq_row + sqc - 2.0 * acc
        d = jnp.where(bi_row != bic, jnp.inf, d)
        dmat_ref[:, pl.ds(c, CT)] = d
        return carry

    jax.lax.fori_loop(0, nt, p1, 0)

    ais = []
    ws = []
    prev_ai = None
    for _ in range(K):
        def step(t, carry, _prev=prev_ai):
            m, a = carry
            c = pl.multiple_of(t * CT, CT)
            dt = dmat_ref[:, pl.ds(c, CT)]
            g = giota_ref[:, pl.ds(pl.multiple_of(c0 + c, CT), CT)]
            if _prev is not None:
                dt = jnp.where(g == _prev, jnp.inf, dt)
                dmat_ref[:, pl.ds(c, CT)] = dt
            mt = jnp.min(dt, axis=1, keepdims=True)
            at = jnp.min(jnp.where(dt == mt, g, BIG), axis=1, keepdims=True)
            a = jnp.where(mt < m, at,
                          jnp.where(mt == m, jnp.minimum(a, at), a))
            return jnp.minimum(m, mt), a

        m, ai = jax.lax.fori_loop(
            0, nt, step, (jnp.full((R, 1), jnp.inf, jnp.float32),
                          jnp.full((R, 1), BIG, jnp.float32)))
        ais.append(ai)
        ws.append(jnp.exp(-10.0 * jnp.maximum(m, 0.0)))
        prev_ai = ai

    idxo_ref[...] = jnp.concatenate(ais, axis=1)
    wo_ref[...] = jnp.concatenate(
        [jnp.broadcast_to(wk, (R, 16)) for wk in ws], axis=1)


def _sc_agg_body(idx_hbm, w_hbm, h_hbm, agg_hbm,
                 idx_v, w_v, rows_v, agg_v, sem, *, rows_w, nch, P):
    wid = lax.axis_index("s") * 2 + lax.axis_index("c")
    row0 = wid * rows_w

    def chunk(ci, carry):
        rbase = row0 + ci * CHR
        ibase = pl.multiple_of(rbase * K, CHR * K)
        pltpu.sync_copy(idx_hbm.at[pl.ds(ibase, CHR * K)], idx_v)
        pltpu.sync_copy(w_hbm.at[pl.ds(ibase, CHR * K)], w_v)
        pltpu.async_copy(h_hbm.at[idx_v], rows_v, sem).wait()
        for n in range(CHR):
            macc = None
            for k in range(K):
                j = n * K + k
                msg = rows_v[j, pl.ds(0, P)] * w_v[j, :]
                if macc is None:
                    macc = msg
                    xacc = msg
                else:
                    macc = macc + msg
                    xacc = jnp.maximum(xacc, msg)
            agg_v[n, pl.ds(0, P)] = macc * (1.0 / K)
            agg_v[n, pl.ds(P, P)] = xacc
        pltpu.sync_copy(agg_v, agg_hbm.at[pl.ds(rbase, CHR)])
        return carry

    jax.lax.fori_loop(0, nch, chunk, 0)


def _out_body(x_ref, agg_ref, wo1_ref, wo2_ref, bo2_ref, gamma_ref, beta_ref,
              out_ref):
    xb = x_ref[...]
    y = (jax.lax.dot_general(_bf(xb), _bf(wo1_ref[...]),
                             (((1,), (0,)), ((), ())),
                             preferred_element_type=jnp.float32)
         + jax.lax.dot_general(_bf(agg_ref[...]), _bf(wo2_ref[...]),
                               (((1,), (0,)), ((), ())),
                               preferred_element_type=jnp.float32)
         + bo2_ref[...] + xb)
    mu = jnp.mean(y, axis=1, keepdims=True)
    var = jnp.mean((y - mu) * (y - mu), axis=1, keepdims=True)
    out_ref[...] = (gamma_ref[...] * (y - mu) / jnp.sqrt(var + 1e-5)
                    + beta_ref[...])


def kernel(x, batch_index, W_s, b_s, W_h, b_h, W_o1, W_o2, b_o2, gamma, beta):
    N, D = x.shape
    S = W_s.shape[1]
    P = W_h.shape[1]
    NPAD = ((N + CT - 1) // CT) * CT
    NB = NPAD // R

    xp = jnp.pad(x, ((0, NPAD - N), (0, 0)))
    bip = jnp.pad(batch_index.astype(jnp.int32), (0, NPAD - N),
                  constant_values=-1)

    s, h, sq = pl.pallas_call(
        _proj_body,
        grid=(NPAD // RO,),
        in_specs=[
            pl.BlockSpec((RO, D), lambda i: (i, 0)),
            pl.BlockSpec((D, S), lambda i: (0, 0)),
            pl.BlockSpec((1, S), lambda i: (0, 0)),
            pl.BlockSpec((D, P), lambda i: (0, 0)),
            pl.BlockSpec((1, P), lambda i: (0, 0)),
        ],
        out_specs=[
            pl.BlockSpec((RO, S), lambda i: (i, 0)),
            pl.BlockSpec((RO, 128), lambda i: (i, 0)),
            pl.BlockSpec((RO, 1), lambda i: (i, 0)),
        ],
        out_shape=[
            jax.ShapeDtypeStruct((NPAD, S), jnp.float32),
            jax.ShapeDtypeStruct((NPAD, 128), jnp.float32),
            jax.ShapeDtypeStruct((NPAD, 1), jnp.float32),
        ],
    )(xp, W_s, b_s.reshape(1, S), W_h, b_h.reshape(1, P))

    sT = s.T
    sqT = sq.reshape(1, NPAD)
    biT = bip.reshape(1, NPAD)
    bi_col = bip.reshape(NPAD, 1)

    blk = jnp.arange(NB, dtype=jnp.int32) * R
    bf = bip[jnp.minimum(blk, N - 1)]
    bl = bip[jnp.minimum(blk + R - 1, N - 1)]
    start = jnp.searchsorted(batch_index, bf, side='left').astype(jnp.int32)
    end = jnp.searchsorted(batch_index, bl, side='right').astype(jnp.int32)
    c0 = (start // CT) * CT
    ntl = (end - c0 + CT - 1) // CT

    grid_spec = pltpu.PrefetchScalarGridSpec(
        num_scalar_prefetch=2,
        grid=(NB,),
        in_specs=[
            pl.BlockSpec((R, S), lambda i, *_: (i, 0)),
            pl.BlockSpec((R, 1), lambda i, *_: (i, 0)),
            pl.BlockSpec((R, 1), lambda i, *_: (i, 0)),
            pl.BlockSpec((S, NPAD), lambda i, *_: (0, 0)),
            pl.BlockSpec((1, NPAD), lambda i, *_: (0, 0)),
            pl.BlockSpec((1, NPAD), lambda i, *_: (0, 0)),
            pl.BlockSpec((1, NPAD), lambda i, *_: (0, 0)),
        ],
        out_specs=[
            pl.BlockSpec((R, K), lambda i, *_: (i, 0)),
            pl.BlockSpec((R, K * 16), lambda i, *_: (i, 0)),
        ],
        scratch_shapes=[pltpu.VMEM((R, NPAD), jnp.float32)],
    )
    giota = jnp.arange(NPAD, dtype=jnp.float32).reshape(1, NPAD)
    idxf, wf = pl.pallas_call(
        functools.partial(_knn_body, S=S),
        grid_spec=grid_spec,
        out_shape=[
            jax.ShapeDtypeStruct((NPAD, K), jnp.float32),
            jax.ShapeDtypeStruct((NPAD, K * 16), jnp.float32),
        ],
    )(c0, ntl, s, sq, bi_col, sT, sqT, biT, giota)

    rows_w = NPAD // NSC
    nch = rows_w // CHR
    idx_flat = idxf.astype(jnp.int32).reshape(NPAD * K)
    w_flat = wf.reshape(NPAD * K, 16)
    mesh = plsc.VectorSubcoreMesh(core_axis_name="c", subcore_axis_name="s")
    agg = pl.kernel(
        functools.partial(_sc_agg_body, rows_w=rows_w, nch=nch, P=P),
        out_type=jax.ShapeDtypeStruct((NPAD, 2 * P), jnp.float32),
        mesh=mesh,
        scratch_types=[
            pltpu.VMEM((CHR * K,), jnp.int32),
            pltpu.VMEM((CHR * K, P), jnp.float32),
            pltpu.VMEM((CHR * K, 128), jnp.float32),
            pltpu.VMEM((CHR, 2 * P), jnp.float32),
            pltpu.SemaphoreType.DMA,
        ],
    )(idx_flat, w_flat, h)

    out = pl.pallas_call(
        _out_body,
        grid=(NPAD // RO,),
        in_specs=[
            pl.BlockSpec((RO, D), lambda i: (i, 0)),
            pl.BlockSpec((RO, 2 * P), lambda i: (i, 0)),
            pl.BlockSpec((D, D), lambda i: (0, 0)),
            pl.BlockSpec((2 * P, D), lambda i: (0, 0)),
            pl.BlockSpec((1, D), lambda i: (0, 0)),
            pl.BlockSpec((1, D), lambda i: (0, 0)),
            pl.BlockSpec((1, D), lambda i: (0, 0)),
        ],
        out_specs=pl.BlockSpec((RO, D), lambda i: (i, 0)),
        out_shape=jax.ShapeDtypeStruct((NPAD, D), jnp.float32),
    )(xp, agg, W_o1, W_o2, b_o2.reshape(1, D), gamma.reshape(1, D),
      beta.reshape(1, D))

    return out[:N]


# speedup vs baseline: 1.1455x; 1.1455x over previous
import functools

import jax
import jax.numpy as jnp
from jax import lax
from jax.experimental import pallas as pl
from jax.experimental.pallas import tpu as pltpu
from jax.experimental.pallas import tpu_sc as plsc

K = 16
R = 640
CT = 512
RO = 512
BIG = 1e9
NSC = 32
CHR = 8


def _bf(v):
    return v.astype(jnp.bfloat16)


def _proj_body(x_ref, ws_ref, bs_ref, wh_ref, bh_ref, s_ref, h_ref, sq_ref):
    xb = _bf(x_ref[...])
    s = jax.lax.dot_general(xb, _bf(ws_ref[...]), (((1,), (0,)), ((), ())),
                            preferred_element_type=jnp.float32) + bs_ref[...]
    h = jax.lax.dot_general(xb, _bf(wh_ref[...]), (((1,), (0,)), ((), ())),
                            preferred_element_type=jnp.float32) + bh_ref[...]
    s_ref[...] = s
    h_ref[...] = jnp.concatenate(
        [h, jnp.zeros((h.shape[0], 128 - h.shape[1]), jnp.float32)], axis=1)
    sq_ref[...] = jnp.sum(s * s, axis=1, keepdims=True)


def _knn_body(c0_ref, nt_ref, srow_ref, sqrow_ref, birow_ref,
              sT_ref, sqT_ref, biT_ref, giota_ref,
              idxo_ref, wo_ref, dmat_ref, *, S):
    i = pl.program_id(0)
    c0 = c0_ref[i]
    nt = nt_ref[i]

    s_row = _bf(srow_ref[...]).astype(jnp.float32)
    sq_row = sqrow_ref[...]
    bi_row = birow_ref[...]

    def p1(t, carry):
        c = pl.multiple_of(t * CT, CT)
        gc = pl.multiple_of(c0 + c, CT)
        scT = _bf(sT_ref[:, pl.ds(gc, CT)]).astype(jnp.float32)
        sqc = sqT_ref[:, pl.ds(gc, CT)]
        bic = biT_ref[:, pl.ds(gc, CT)]
        acc = s_row[:, 0:1] * scT[0:1, :]
        for a in range(1, S):
            acc = acc + s_row[:, a:a + 1] * scT[a:a + 1, :]
        d = sq_row + sqc - 2.0 * acc
        d = jnp.where(bi_row != bic, jnp.inf, d)
        dmat_ref[:, pl.ds(c, CT)] = d
        return carry

    jax.lax.fori_loop(0, nt, p1, 0)

    ais = []
    ws = []
    prev_ai = None
    for _ in range(K):
        def step(t, carry, _prev=prev_ai):
            m, a = carry
            c = pl.multiple_of(t * CT, CT)
            dt = dmat_ref[:, pl.ds(c, CT)]
            g = giota_ref[:, pl.ds(pl.multiple_of(c0 + c, CT), CT)]
            if _prev is not None:
                dt = jnp.where(g == _prev, jnp.inf, dt)
                dmat_ref[:, pl.ds(c, CT)] = dt
            mt = jnp.min(dt, axis=1, keepdims=True)
            at = jnp.min(jnp.where(dt == mt, g, BIG), axis=1, keepdims=True)
            a = jnp.where(mt < m, at,
                          jnp.where(mt == m, jnp.minimum(a, at), a))
            return jnp.minimum(m, mt), a

        m, ai = jax.lax.fori_loop(
            0, nt, step, (jnp.full((R, 1), jnp.inf, jnp.float32),
                          jnp.full((R, 1), BIG, jnp.float32)))
        ais.append(ai)
        ws.append(jnp.exp(-10.0 * jnp.maximum(m, 0.0)))
        prev_ai = ai

    idxo_ref[...] = jnp.concatenate(ais, axis=1)
    wo_ref[...] = jnp.concatenate(
        [jnp.broadcast_to(wk, (R, 16)) for wk in ws], axis=1)


def _sc_agg_body(idx_hbm, w_hbm, h_hbm, agg_hbm,
                 idx_v, w_v, rows_v, agg_v, sem, *, rows_w, nch, P):
    wid = lax.axis_index("s") * 2 + lax.axis_index("c")
    row0 = wid * rows_w

    ibase0 = pl.multiple_of(row0 * K, rows_w * K)
    pltpu.sync_copy(idx_hbm.at[pl.ds(ibase0, rows_w * K)], idx_v)
    pltpu.sync_copy(w_hbm.at[pl.ds(ibase0 * P, rows_w * K * P)], w_v)

    def chunk(ci, carry):
        off = pl.multiple_of(ci * (CHR * K), CHR * K)
        woff = pl.multiple_of(ci * (CHR * K * P), CHR * K * P)
        aoff = pl.multiple_of(ci * (CHR * 2 * P), CHR * 2 * P)
        pltpu.async_copy(h_hbm.at[idx_v.at[pl.ds(off, CHR * K)]],
                         rows_v, sem).wait()
        for n in range(CHR):
            macc = None
            for k in range(K):
                j = n * K + k
                wv = w_v[pl.ds(woff + j * P, P)]
                msg = rows_v[j, pl.ds(0, P)] * wv
                if macc is None:
                    macc = msg
                    xacc = msg
                else:
                    macc = macc + msg
                    xacc = jnp.maximum(xacc, msg)
            agg_v[pl.ds(aoff + n * 2 * P, P)] = macc * (1.0 / K)
            agg_v[pl.ds(aoff + n * 2 * P + P, P)] = xacc
        return carry

    jax.lax.fori_loop(0, nch, chunk, 0)
    pltpu.sync_copy(agg_v, agg_hbm.at[pl.ds(row0 * 2 * P, rows_w * 2 * P)])


def _out_body(x_ref, agg_ref, wo1_ref, wo2_ref, bo2_ref, gamma_ref, beta_ref,
              out_ref):
    xb = x_ref[...]
    y = (jax.lax.dot_general(_bf(xb), _bf(wo1_ref[...]),
                             (((1,), (0,)), ((), ())),
                             preferred_element_type=jnp.float32)
         + jax.lax.dot_general(_bf(agg_ref[...]), _bf(wo2_ref[...]),
                               (((1,), (0,)), ((), ())),
                               preferred_element_type=jnp.float32)
         + bo2_ref[...] + xb)
    mu = jnp.mean(y, axis=1, keepdims=True)
    var = jnp.mean((y - mu) * (y - mu), axis=1, keepdims=True)
    out_ref[...] = (gamma_ref[...] * (y - mu) / jnp.sqrt(var + 1e-5)
                    + beta_ref[...])


def kernel(x, batch_index, W_s, b_s, W_h, b_h, W_o1, W_o2, b_o2, gamma, beta):
    N, D = x.shape
    S = W_s.shape[1]
    P = W_h.shape[1]
    NPAD = ((N + CT - 1) // CT) * CT
    NB = NPAD // R

    xp = jnp.pad(x, ((0, NPAD - N), (0, 0)))
    bip = jnp.pad(batch_index.astype(jnp.int32), (0, NPAD - N),
                  constant_values=-1)

    s, h, sq = pl.pallas_call(
        _proj_body,
        grid=(NPAD // RO,),
        in_specs=[
            pl.BlockSpec((RO, D), lambda i: (i, 0)),
            pl.BlockSpec((D, S), lambda i: (0, 0)),
            pl.BlockSpec((1, S), lambda i: (0, 0)),
            pl.BlockSpec((D, P), lambda i: (0, 0)),
            pl.BlockSpec((1, P), lambda i: (0, 0)),
        ],
        out_specs=[
            pl.BlockSpec((RO, S), lambda i: (i, 0)),
            pl.BlockSpec((RO, 128), lambda i: (i, 0)),
            pl.BlockSpec((RO, 1), lambda i: (i, 0)),
        ],
        out_shape=[
            jax.ShapeDtypeStruct((NPAD, S), jnp.float32),
            jax.ShapeDtypeStruct((NPAD, 128), jnp.float32),
            jax.ShapeDtypeStruct((NPAD, 1), jnp.float32),
        ],
    )(xp, W_s, b_s.reshape(1, S), W_h, b_h.reshape(1, P))

    sT = s.T
    sqT = sq.reshape(1, NPAD)
    biT = bip.reshape(1, NPAD)
    bi_col = bip.reshape(NPAD, 1)

    blk = jnp.arange(NB, dtype=jnp.int32) * R
    bf = bip[jnp.minimum(blk, N - 1)]
    bl = bip[jnp.minimum(blk + R - 1, N - 1)]
    start = jnp.searchsorted(batch_index, bf, side='left').astype(jnp.int32)
    end = jnp.searchsorted(batch_index, bl, side='right').astype(jnp.int32)
    c0 = (start // CT) * CT
    ntl = (end - c0 + CT - 1) // CT

    grid_spec = pltpu.PrefetchScalarGridSpec(
        num_scalar_prefetch=2,
        grid=(NB,),
        in_specs=[
            pl.BlockSpec((R, S), lambda i, *_: (i, 0)),
            pl.BlockSpec((R, 1), lambda i, *_: (i, 0)),
            pl.BlockSpec((R, 1), lambda i, *_: (i, 0)),
            pl.BlockSpec((S, NPAD), lambda i, *_: (0, 0)),
            pl.BlockSpec((1, NPAD), lambda i, *_: (0, 0)),
            pl.BlockSpec((1, NPAD), lambda i, *_: (0, 0)),
            pl.BlockSpec((1, NPAD), lambda i, *_: (0, 0)),
        ],
        out_specs=[
            pl.BlockSpec((R, K), lambda i, *_: (i, 0)),
            pl.BlockSpec((R, K * 16), lambda i, *_: (i, 0)),
        ],
        scratch_shapes=[pltpu.VMEM((R, NPAD), jnp.float32)],
    )
    giota = jnp.arange(NPAD, dtype=jnp.float32).reshape(1, NPAD)
    idxf, wf = pl.pallas_call(
        functools.partial(_knn_body, S=S),
        grid_spec=grid_spec,
        out_shape=[
            jax.ShapeDtypeStruct((NPAD, K), jnp.float32),
            jax.ShapeDtypeStruct((NPAD, K * 16), jnp.float32),
        ],
    )(c0, ntl, s, sq, bi_col, sT, sqT, biT, giota)

    rows_w = NPAD // NSC
    nch = rows_w // CHR
    idx_flat = idxf.astype(jnp.int32).reshape(NPAD * K)
    w_flat = wf.reshape(NPAD * K * 16)
    mesh = plsc.VectorSubcoreMesh(core_axis_name="c", subcore_axis_name="s")
    agg = pl.kernel(
        functools.partial(_sc_agg_body, rows_w=rows_w, nch=nch, P=P),
        out_type=jax.ShapeDtypeStruct((NPAD * 2 * P,), jnp.float32),
        mesh=mesh,
        scratch_types=[
            pltpu.VMEM((rows_w * K,), jnp.int32),
            pltpu.VMEM((rows_w * K * P,), jnp.float32),
            pltpu.VMEM((CHR * K, 128), jnp.float32),
            pltpu.VMEM((rows_w * 2 * P,), jnp.float32),
            pltpu.SemaphoreType.DMA,
        ],
    )(idx_flat, w_flat, h)
    agg = agg.reshape(NPAD, 2 * P)

    out = pl.pallas_call(
        _out_body,
        grid=(NPAD // RO,),
        in_specs=[
            pl.BlockSpec((RO, D), lambda i: (i, 0)),
            pl.BlockSpec((RO, 2 * P), lambda i: (i, 0)),
            pl.BlockSpec((D, D), lambda i: (0, 0)),
            pl.BlockSpec((2 * P, D), lambda i: (0, 0)),
            pl.BlockSpec((1, D), lambda i: (0, 0)),
            pl.BlockSpec((1, D), lambda i: (0, 0)),
            pl.BlockSpec((1, D), lambda i: (0, 0)),
        ],
        out_specs=pl.BlockSpec((RO, D), lambda i: (i, 0)),
        out_shape=jax.ShapeDtypeStruct((NPAD, D), jnp.float32),
    )(xp, agg, W_o1, W_o2, b_o2.reshape(1, D), gamma.reshape(1, D),
      beta.reshape(1, D))

    return out[:N]


# speedup vs baseline: 1.1568x; 1.0098x over previous
import functools

import jax
import jax.numpy as jnp
from jax import lax
from jax.experimental import pallas as pl
from jax.experimental.pallas import tpu as pltpu
from jax.experimental.pallas import tpu_sc as plsc

K = 16
R = 640
CT = 512
RO = 512
BIG = 1e9
NSC = 32
CHR = 8


def _bf(v):
    return v.astype(jnp.bfloat16)


def _proj_body(x_ref, ws_ref, bs_ref, wh_ref, bh_ref, s_ref, h_ref, sq_ref):
    xb = _bf(x_ref[...])
    s = jax.lax.dot_general(xb, _bf(ws_ref[...]), (((1,), (0,)), ((), ())),
                            preferred_element_type=jnp.float32) + bs_ref[...]
    h = jax.lax.dot_general(xb, _bf(wh_ref[...]), (((1,), (0,)), ((), ())),
                            preferred_element_type=jnp.float32) + bh_ref[...]
    s_ref[...] = s
    h_ref[...] = jnp.concatenate(
        [h, jnp.zeros((h.shape[0], 128 - h.shape[1]), jnp.float32)], axis=1)
    sq_ref[...] = jnp.sum(s * s, axis=1, keepdims=True)


def _knn_body(c0_ref, nt_ref, srow_ref, sqrow_ref, birow_ref,
              sT_ref, sqT_ref, biT_ref, giota_ref,
              idxo_ref, wo_ref, dmat_ref, *, S):
    i = pl.program_id(0)
    c0 = c0_ref[i]
    nt = nt_ref[i]

    s_row = _bf(srow_ref[...]).astype(jnp.float32)
    sq_row = sqrow_ref[...]
    bi_row = birow_ref[...]

    def p1(t, carry):
        c = pl.multiple_of(t * CT, CT)
        gc = pl.multiple_of(c0 + c, CT)
        scT = _bf(sT_ref[:, pl.ds(gc, CT)]).astype(jnp.float32)
        sqc = sqT_ref[:, pl.ds(gc, CT)]
        bic = biT_ref[:, pl.ds(gc, CT)]
        acc = s_row[:, 0:1] * scT[0:1, :]
        for a in range(1, S):
            acc = acc + s_row[:, a:a + 1] * scT[a:a + 1, :]
        d = sq_row + sqc - 2.0 * acc
        d = jnp.where(bi_row != bic, jnp.inf, d)
        dmat_ref[:, pl.ds(c, CT)] = d
        return carry

    jax.lax.fori_loop(0, nt, p1, 0)

    ais = []
    ws = []
    prev_ai = None
    for _ in range(K):
        def step(t, carry, _prev=prev_ai):
            m, a = carry
            c = pl.multiple_of(t * CT, CT)
            dt = dmat_ref[:, pl.ds(c, CT)]
            g = giota_ref[:, pl.ds(pl.multiple_of(c0 + c, CT), CT)]
            if _prev is not None:
                dt = jnp.where(g == _prev, jnp.inf, dt)
                dmat_ref[:, pl.ds(c, CT)] = dt
            mt = jnp.min(dt, axis=1, keepdims=True)
            at = jnp.min(jnp.where(dt == mt, g, BIG), axis=1, keepdims=True)
            a = jnp.where(mt < m, at,
                          jnp.where(mt == m, jnp.minimum(a, at), a))
            return jnp.minimum(m, mt), a

        m, ai = jax.lax.fori_loop(
            0, nt, step, (jnp.full((R, 1), jnp.inf, jnp.float32),
                          jnp.full((R, 1), BIG, jnp.float32)))
        ais.append(ai)
        ws.append(jnp.exp(-10.0 * jnp.maximum(m, 0.0)))
        prev_ai = ai

    idxo_ref[...] = jnp.concatenate(ais, axis=1)
    wo_ref[...] = jnp.concatenate(
        [jnp.broadcast_to(wk, (R, 16)) for wk in ws], axis=1)


def _sc_agg_body(idx_hbm, w_hbm, h_hbm, agg_hbm,
                 idx_v, w_v, rows_v, agg_v, sem, *, rows_w, nch, P):
    wid = lax.axis_index("s") * 2 + lax.axis_index("c")
    row0 = wid * rows_w

    ibase0 = pl.multiple_of(row0 * K, rows_w * K)
    pltpu.sync_copy(idx_hbm.at[pl.ds(ibase0, rows_w * K)], idx_v)
    pltpu.sync_copy(w_hbm.at[pl.ds(row0, rows_w)], w_v)

    def chunk(ci, carry):
        off = pl.multiple_of(ci * (CHR * K), CHR * K)
        roff = pl.multiple_of(ci * CHR, CHR)
        aoff = pl.multiple_of(ci * (CHR * 2 * P), CHR * 2 * P)
        pltpu.async_copy(h_hbm.at[idx_v.at[pl.ds(off, CHR * K)]],
                         rows_v, sem).wait()
        for n in range(CHR):
            macc = None
            for k in range(K):
                j = n * K + k
                wv = w_v[pl.ds(roff + n, 1), pl.ds(k * P, P)].reshape(P)
                msg = rows_v[j, pl.ds(0, P)] * wv
                if macc is None:
                    macc = msg
                    xacc = msg
                else:
                    macc = macc + msg
                    xacc = jnp.maximum(xacc, msg)
            agg_v[pl.ds(aoff + n * 2 * P, P)] = macc * (1.0 / K)
            agg_v[pl.ds(aoff + n * 2 * P + P, P)] = xacc
        return carry

    jax.lax.fori_loop(0, nch, chunk, 0)
    pltpu.sync_copy(agg_v, agg_hbm.at[pl.ds(row0 * 2 * P, rows_w * 2 * P)])


def _out_body(x_ref, agg_ref, wo1_ref, wo2_ref, bo2_ref, gamma_ref, beta_ref,
              out_ref):
    xb = x_ref[...]
    y = (jax.lax.dot_general(_bf(xb), _bf(wo1_ref[...]),
                             (((1,), (0,)), ((), ())),
                             preferred_element_type=jnp.float32)
         + jax.lax.dot_general(_bf(agg_ref[...]), _bf(wo2_ref[...]),
                               (((1,), (0,)), ((), ())),
                               preferred_element_type=jnp.float32)
         + bo2_ref[...] + xb)
    mu = jnp.mean(y, axis=1, keepdims=True)
    var = jnp.mean((y - mu) * (y - mu), axis=1, keepdims=True)
    out_ref[...] = (gamma_ref[...] * (y - mu) / jnp.sqrt(var + 1e-5)
                    + beta_ref[...])


def kernel(x, batch_index, W_s, b_s, W_h, b_h, W_o1, W_o2, b_o2, gamma, beta):
    N, D = x.shape
    S = W_s.shape[1]
    P = W_h.shape[1]
    NPAD = ((N + CT - 1) // CT) * CT
    NB = NPAD // R

    xp = jnp.pad(x, ((0, NPAD - N), (0, 0)))
    bip = jnp.pad(batch_index.astype(jnp.int32), (0, NPAD - N),
                  constant_values=-1)

    s, h, sq = pl.pallas_call(
        _proj_body,
        grid=(NPAD // RO,),
        in_specs=[
            pl.BlockSpec((RO, D), lambda i: (i, 0)),
            pl.BlockSpec((D, S), lambda i: (0, 0)),
            pl.BlockSpec((1, S), lambda i: (0, 0)),
            pl.BlockSpec((D, P), lambda i: (0, 0)),
            pl.BlockSpec((1, P), lambda i: (0, 0)),
        ],
        out_specs=[
            pl.BlockSpec((RO, S), lambda i: (i, 0)),
            pl.BlockSpec((RO, 128), lambda i: (i, 0)),
            pl.BlockSpec((RO, 1), lambda i: (i, 0)),
        ],
        out_shape=[
            jax.ShapeDtypeStruct((NPAD, S), jnp.float32),
            jax.ShapeDtypeStruct((NPAD, 128), jnp.float32),
            jax.ShapeDtypeStruct((NPAD, 1), jnp.float32),
        ],
    )(xp, W_s, b_s.reshape(1, S), W_h, b_h.reshape(1, P))

    sT = s.T
    sqT = sq.reshape(1, NPAD)
    biT = bip.reshape(1, NPAD)
    bi_col = bip.reshape(NPAD, 1)

    blk = jnp.arange(NB, dtype=jnp.int32) * R
    bf = bip[jnp.minimum(blk, N - 1)]
    bl = bip[jnp.minimum(blk + R - 1, N - 1)]
    start = jnp.searchsorted(batch_index, bf, side='left').astype(jnp.int32)
    end = jnp.searchsorted(batch_index, bl, side='right').astype(jnp.int32)
    c0 = (start // CT) * CT
    ntl = (end - c0 + CT - 1) // CT

    grid_spec = pltpu.PrefetchScalarGridSpec(
        num_scalar_prefetch=2,
        grid=(NB,),
        in_specs=[
            pl.BlockSpec((R, S), lambda i, *_: (i, 0)),
            pl.BlockSpec((R, 1), lambda i, *_: (i, 0)),
            pl.BlockSpec((R, 1), lambda i, *_: (i, 0)),
            pl.BlockSpec((S, NPAD), lambda i, *_: (0, 0)),
            pl.BlockSpec((1, NPAD), lambda i, *_: (0, 0)),
            pl.BlockSpec((1, NPAD), lambda i, *_: (0, 0)),
            pl.BlockSpec((1, NPAD), lambda i, *_: (0, 0)),
        ],
        out_specs=[
            pl.BlockSpec((R, K), lambda i, *_: (i, 0)),
            pl.BlockSpec((R, K * 16), lambda i, *_: (i, 0)),
        ],
        scratch_shapes=[pltpu.VMEM((R, NPAD), jnp.float32)],
    )
    giota = jnp.arange(NPAD, dtype=jnp.float32).reshape(1, NPAD)
    idxf, wf = pl.pallas_call(
        functools.partial(_knn_body, S=S),
        grid_spec=grid_spec,
        out_shape=[
            jax.ShapeDtypeStruct((NPAD, K), jnp.float32),
            jax.ShapeDtypeStruct((NPAD, K * 16), jnp.float32),
        ],
    )(c0, ntl, s, sq, bi_col, sT, sqT, biT, giota)

    rows_w = NPAD // NSC
    nch = rows_w // CHR
    idx_flat = idxf.astype(jnp.int32).reshape(NPAD * K)
    w_flat = wf
    mesh = plsc.VectorSubcoreMesh(core_axis_name="c", subcore_axis_name="s")
    agg = pl.kernel(
        functools.partial(_sc_agg_body, rows_w=rows_w, nch=nch, P=P),
        out_type=jax.ShapeDtypeStruct((NPAD * 2 * P,), jnp.float32),
        mesh=mesh,
        scratch_types=[
            pltpu.VMEM((rows_w * K,), jnp.int32),
            pltpu.VMEM((rows_w, K * P), jnp.float32),
            pltpu.VMEM((CHR * K, 128), jnp.float32),
            pltpu.VMEM((rows_w * 2 * P,), jnp.float32),
            pltpu.SemaphoreType.DMA,
        ],
    )(idx_flat, w_flat, h)
    agg = agg.reshape(NPAD, 2 * P)

    out = pl.pallas_call(
        _out_body,
        grid=(NPAD // RO,),
        in_specs=[
            pl.BlockSpec((RO, D), lambda i: (i, 0)),
            pl.BlockSpec((RO, 2 * P), lambda i: (i, 0)),
            pl.BlockSpec((D, D), lambda i: (0, 0)),
            pl.BlockSpec((2 * P, D), lambda i: (0, 0)),
            pl.BlockSpec((1, D), lambda i: (0, 0)),
            pl.BlockSpec((1, D), lambda i: (0, 0)),
            pl.BlockSpec((1, D), lambda i: (0, 0)),
        ],
        out_specs=pl.BlockSpec((RO, D), lambda i: (i, 0)),
        out_shape=jax.ShapeDtypeStruct((NPAD, D), jnp.float32),
    )(xp, agg, W_o1, W_o2, b_o2.reshape(1, D), gamma.reshape(1, D),
      beta.reshape(1, D))

    return out[:N]
